# baseline (speedup 1.0000x reference)
import functools

import jax
import jax.numpy as jnp
from jax import lax
from jax.experimental import pallas as pl
from jax.experimental.pallas import tpu as pltpu
from jax.experimental.pallas import tpu_sc as plsc

N = 10000
E = 320000
D = 128
C = 64
DH = D // 2
NP = 10240
STRIPE = NP // 16
RB = 512
GRID = NP // RB

DEG_CH = 2000
DEG_PER_W = E // 32
MSG_CH = 320
MSG_NCH = 32
EPAD = 2 * 16 * MSG_NCH * MSG_CH


def _sc_mesh():
    return plsc.VectorSubcoreMesh(core_axis_name="c", subcore_axis_name="s")


def _deg_body(dst_hbm, ones_hbm, zrow_hbm, out_hbm, idx_v, ones_v, deg_s):
    c = lax.axis_index("c")
    s = lax.axis_index("s")
    pltpu.sync_copy(zrow_hbm, deg_s.at[pl.ds(s * STRIPE, STRIPE)])
    pltpu.sync_copy(ones_hbm, ones_v)
    plsc.subcore_barrier()
    base = (s * 2 + c) * DEG_PER_W
    for k in range(DEG_PER_W // DEG_CH):
        pltpu.sync_copy(dst_hbm.at[pl.ds(base + k * DEG_CH, DEG_CH)], idx_v)
        pltpu.sync_copy(ones_v, deg_s.at[idx_v], add=True)
    plsc.subcore_barrier()
    pltpu.sync_copy(deg_s.at[pl.ds(s * STRIPE, STRIPE)],
                    out_hbm.at[c, pl.ds(s * STRIPE, STRIPE)])


def _deg_call(dst, ones_ch, zrow):
    f = pl.kernel(
        _deg_body,
        out_type=jax.ShapeDtypeStruct((2, NP), jnp.float32),
        mesh=_sc_mesh(),
        scratch_types=[
            pltpu.VMEM((DEG_CH,), jnp.int32),
            pltpu.VMEM((DEG_CH,), jnp.float32),
            pltpu.VMEM_SHARED((NP,), jnp.float32),
        ],
    )
    return f(dst, ones_ch, zrow)


def _msg_body(y_hbm, src_hbm, dst_hbm, zblk_hbm, out_hbm,
              isrc_v, idst_v, rows_v, acc_s, sem):
    c = lax.axis_index("c")
    s = lax.axis_index("s")
    pltpu.sync_copy(zblk_hbm, acc_s.at[pl.ds(s * STRIPE, STRIPE), :])
    plsc.subcore_barrier()
    ebase = (c * 16 + s) * (MSG_NCH * MSG_CH)

    def chunk(k, carry):
        off = ebase + k * MSG_CH
        pltpu.sync_copy(src_hbm.at[pl.ds(off, MSG_CH)], isrc_v)
        pltpu.sync_copy(dst_hbm.at[pl.ds(off, MSG_CH)], idst_v)
        pltpu.async_copy(y_hbm.at[isrc_v], rows_v, sem).wait()
        pltpu.sync_copy(rows_v, acc_s.at[idst_v], add=True)
        return carry

    lax.fori_loop(0, MSG_NCH, chunk, 0)
    plsc.subcore_barrier()
    pltpu.sync_copy(acc_s.at[pl.ds(s * STRIPE, STRIPE), :],
                    out_hbm.at[c, pl.ds(s * STRIPE, STRIPE), :])


def _msg_call(y, src, dst, zblk):
    f = pl.kernel(
        _msg_body,
        out_type=jax.ShapeDtypeStruct((2, NP, D), jnp.float32),
        mesh=_sc_mesh(),
        scratch_types=[
            pltpu.VMEM((MSG_CH,), jnp.int32),
            pltpu.VMEM((MSG_CH,), jnp.int32),
            pltpu.VMEM((MSG_CH, D), jnp.float32),
            pltpu.VMEM_SHARED((NP, D), jnp.float32),
            pltpu.SemaphoreType.DMA,
        ],
    )
    return f(y, src, dst, zblk)


def _tc1_body(x_ref, w_ref, dega_ref, degb_ref, y_ref):
    xw = jnp.dot(x_ref[...], w_ref[...], preferred_element_type=jnp.float32)
    deg = 1.0 + (dega_ref[...] + degb_ref[...]).reshape(RB)
    dinv = lax.rsqrt(deg)
    y_ref[...] = xw * dinv[:, None]


def _tc1_call(xp, W1, dega, degb):
    return pl.pallas_call(
        _tc1_body,
        grid=(GRID,),
        in_specs=[
            pl.BlockSpec((RB, D), lambda i: (i, 0)),
            pl.BlockSpec((D, D), lambda i: (0, 0)),
            pl.BlockSpec((1, 1, RB), lambda i: (i, 0, 0)),
            pl.BlockSpec((1, 1, RB), lambda i: (i, 0, 0)),
        ],
        out_specs=pl.BlockSpec((RB, D), lambda i: (i, 0)),
        out_shape=jax.ShapeDtypeStruct((NP, D), jnp.float32),
    )(xp, W1, dega, degb)


def _tc2_body(acc_ref, y_ref, x_ref, dega_ref, degb_ref, b1_ref, g_ref,
              be_ref, wfc_ref, bfc_ref, o_ref):
    acc = acc_ref[0] + acc_ref[1] + y_ref[...]
    deg = 1.0 + (dega_ref[...] + degb_ref[...]).reshape(RB)
    dinv = lax.rsqrt(deg)
    x = x_ref[...]
    h = b1_ref[...] + acc * dinv[:, None]
    h = jnp.maximum(h, 0.0) * x
    mean = jnp.mean(h, axis=-1, keepdims=True)
    var = jnp.mean((h - mean) ** 2, axis=-1, keepdims=True)
    h = (h - mean) * lax.rsqrt(var + 1e-5) * g_ref[...] + be_ref[...]
    h = h + x
    o_ref[...] = jnp.dot(h, wfc_ref[...],
                         preferred_element_type=jnp.float32) + bfc_ref[...]


def _tc2_call(acc2, y2, xp, dega, degb, b1, g, be, Wfc, bfc):
    return pl.pallas_call(
        _tc2_body,
        grid=(GRID,),
        in_specs=[
            pl.BlockSpec((2, RB, D), lambda i: (0, i, 0)),
            pl.BlockSpec((RB, D), lambda i: (i, 0)),
            pl.BlockSpec((RB, D), lambda i: (i, 0)),
            pl.BlockSpec((1, 1, RB), lambda i: (i, 0, 0)),
            pl.BlockSpec((1, 1, RB), lambda i: (i, 0, 0)),
            pl.BlockSpec((1, D), lambda i: (0, 0)),
            pl.BlockSpec((1, D), lambda i: (0, 0)),
            pl.BlockSpec((1, D), lambda i: (0, 0)),
            pl.BlockSpec((D, C), lambda i: (0, 0)),
            pl.BlockSpec((1, C), lambda i: (0, 0)),
        ],
        out_specs=pl.BlockSpec((RB, C), lambda i: (i, 0)),
        out_shape=jax.ShapeDtypeStruct((NP, C), jnp.float32),
    )(acc2, y2, xp, dega, degb, b1, g, be, Wfc, bfc)


def kernel(x, edge_index, W1, b1, ln_gamma, ln_beta, Wfc, bfc):
    src = edge_index[0]
    dst = edge_index[1]
    ones_ch = jnp.ones((DEG_CH,), jnp.float32)
    zrow = jnp.zeros((STRIPE,), jnp.float32)
    zblk = jnp.zeros((STRIPE, D), jnp.float32)
    xp = jnp.pad(x, ((0, NP - N), (0, 0)))

    src_p = jnp.concatenate([src, jnp.zeros((EPAD - E,), jnp.int32)])
    dst_p = jnp.concatenate([dst, jnp.full((EPAD - E,), NP - 1, jnp.int32)])

    deg2 = _deg_call(dst, ones_ch, zrow)
    dega = deg2[0].reshape(GRID, 1, RB)
    degb = deg2[1].reshape(GRID, 1, RB)
    y = _tc1_call(xp, W1, dega, degb)
    acc2 = _msg_call(y, src_p, dst_p, zblk)
    out = _tc2_call(acc2, y, xp, dega, degb,
                    b1.reshape(1, D), ln_gamma.reshape(1, D),
                    ln_beta.reshape(1, D), Wfc, bfc.reshape(1, C))
    return out[:N]


# speedup vs baseline: 2.2518x; 2.2518x over previous
import functools

import jax
import jax.numpy as jnp
from jax import lax
from jax.experimental import pallas as pl
from jax.experimental.pallas import tpu as pltpu
from jax.experimental.pallas import tpu_sc as plsc

N = 10000
E = 320000
D = 128
C = 64
DH = D // 2
NP = 10240
STRIPE = NP // 16
RB = 512
GRID = NP // RB

DEG_CH = 2000
DEG_PER_W = E // 32
MSG_CH = 320
MSG_NCH = 32
EPAD = 2 * 16 * MSG_NCH * MSG_CH


def _sc_mesh():
    return plsc.VectorSubcoreMesh(core_axis_name="c", subcore_axis_name="s")


def _deg_body(dst_hbm, ones_hbm, zrow_hbm, out_hbm, idx_v, ones_v, deg_s):
    c = lax.axis_index("c")
    s = lax.axis_index("s")
    pltpu.sync_copy(zrow_hbm, deg_s.at[pl.ds(s * STRIPE, STRIPE)])
    pltpu.sync_copy(ones_hbm, ones_v)
    plsc.subcore_barrier()
    base = (s * 2 + c) * DEG_PER_W
    for k in range(DEG_PER_W // DEG_CH):
        pltpu.sync_copy(dst_hbm.at[pl.ds(base + k * DEG_CH, DEG_CH)], idx_v)
        pltpu.sync_copy(ones_v, deg_s.at[idx_v], add=True)
    plsc.subcore_barrier()
    pltpu.sync_copy(deg_s.at[pl.ds(s * STRIPE, STRIPE)],
                    out_hbm.at[c, pl.ds(s * STRIPE, STRIPE)])


def _deg_call(dst, ones_ch, zrow):
    f = pl.kernel(
        _deg_body,
        out_type=jax.ShapeDtypeStruct((2, NP), jnp.float32),
        mesh=_sc_mesh(),
        scratch_types=[
            pltpu.VMEM((DEG_CH,), jnp.int32),
            pltpu.VMEM((DEG_CH,), jnp.float32),
            pltpu.VMEM_SHARED((NP,), jnp.float32),
        ],
    )
    return f(dst, ones_ch, zrow)


def _msg_body(y_hbm, src_hbm, dst_hbm, zblk_hbm, out_hbm,
              isrc_v, idst_v, rows_v, acc_s, sem):
    c = lax.axis_index("c")
    s = lax.axis_index("s")
    pltpu.sync_copy(zblk_hbm, acc_s.at[pl.ds(s * STRIPE, STRIPE), :])
    plsc.subcore_barrier()
    ebase = (c * 16 + s) * (MSG_NCH * MSG_CH)

    def chunk(k, carry):
        off = ebase + k * MSG_CH
        pltpu.sync_copy(src_hbm.at[pl.ds(off, MSG_CH)], isrc_v)
        pltpu.sync_copy(dst_hbm.at[pl.ds(off, MSG_CH)], idst_v)
        pltpu.async_copy(y_hbm.at[isrc_v], rows_v, sem).wait()
        pltpu.sync_copy(rows_v, acc_s.at[idst_v], add=True)
        return carry

    lax.fori_loop(0, MSG_NCH, chunk, 0)
    plsc.subcore_barrier()
    pltpu.sync_copy(acc_s.at[pl.ds(s * STRIPE, STRIPE), :],
                    out_hbm.at[c, pl.ds(s * STRIPE, STRIPE), :])


def _msg_call(y, src, dst, zblk):
    f = pl.kernel(
        _msg_body,
        out_type=jax.ShapeDtypeStruct((2, NP, D), jnp.float32),
        mesh=_sc_mesh(),
        scratch_types=[
            pltpu.VMEM((MSG_CH,), jnp.int32),
            pltpu.VMEM((MSG_CH,), jnp.int32),
            pltpu.VMEM((MSG_CH, D), jnp.float32),
            pltpu.VMEM_SHARED((NP, D), jnp.float32),
            pltpu.SemaphoreType.DMA,
        ],
    )
    return f(y, src, dst, zblk)


def _tc1_body(x_ref, w_ref, dega_ref, degb_ref, y_ref):
    xw = jnp.dot(x_ref[...], w_ref[...], preferred_element_type=jnp.float32)
    deg = 1.0 + (dega_ref[...] + degb_ref[...]).reshape(RB)
    dinv = lax.rsqrt(deg)
    y_ref[...] = xw * dinv[:, None]


def _tc1_call(xp, W1, dega, degb):
    return pl.pallas_call(
        _tc1_body,
        grid=(GRID,),
        in_specs=[
            pl.BlockSpec((RB, D), lambda i: (i, 0)),
            pl.BlockSpec((D, D), lambda i: (0, 0)),
            pl.BlockSpec((1, 1, RB), lambda i: (i, 0, 0)),
            pl.BlockSpec((1, 1, RB), lambda i: (i, 0, 0)),
        ],
        out_specs=pl.BlockSpec((RB, D), lambda i: (i, 0)),
        out_shape=jax.ShapeDtypeStruct((NP, D), jnp.float32),
    )(xp, W1, dega, degb)


def _tc2_body(acc_ref, y_ref, x_ref, dega_ref, degb_ref, b1_ref, g_ref,
              be_ref, wfc_ref, bfc_ref, o_ref):
    acc = acc_ref[0] + acc_ref[1] + y_ref[...]
    deg = 1.0 + (dega_ref[...] + degb_ref[...]).reshape(RB)
    dinv = lax.rsqrt(deg)
    x = x_ref[...]
    h = b1_ref[...] + acc * dinv[:, None]
    h = jnp.maximum(h, 0.0) * x
    mean = jnp.mean(h, axis=-1, keepdims=True)
    var = jnp.mean((h - mean) ** 2, axis=-1, keepdims=True)
    h = (h - mean) * lax.rsqrt(var + 1e-5) * g_ref[...] + be_ref[...]
    h = h + x
    o_ref[...] = jnp.dot(h, wfc_ref[...],
                         preferred_element_type=jnp.float32) + bfc_ref[...]


def _tc2_call(acc2, y2, xp, dega, degb, b1, g, be, Wfc, bfc):
    return pl.pallas_call(
        _tc2_body,
        grid=(GRID,),
        in_specs=[
            pl.BlockSpec((2, RB, D), lambda i: (0, i, 0)),
            pl.BlockSpec((RB, D), lambda i: (i, 0)),
            pl.BlockSpec((RB, D), lambda i: (i, 0)),
            pl.BlockSpec((1, 1, RB), lambda i: (i, 0, 0)),
            pl.BlockSpec((1, 1, RB), lambda i: (i, 0, 0)),
            pl.BlockSpec((1, D), lambda i: (0, 0)),
            pl.BlockSpec((1, D), lambda i: (0, 0)),
            pl.BlockSpec((1, D), lambda i: (0, 0)),
            pl.BlockSpec((D, C), lambda i: (0, 0)),
            pl.BlockSpec((1, C), lambda i: (0, 0)),
        ],
        out_specs=pl.BlockSpec((RB, C), lambda i: (i, 0)),
        out_shape=jax.ShapeDtypeStruct((NP, C), jnp.float32),
    )(acc2, y2, xp, dega, degb, b1, g, be, Wfc, bfc)


def kernel(x, edge_index, W1, b1, ln_gamma, ln_beta, Wfc, bfc):
    src = edge_index[0]
    dst = edge_index[1]
    ones_ch = jnp.ones((DEG_CH,), jnp.float32)
    zrow = jnp.zeros((STRIPE,), jnp.float32)
    zblk = jnp.zeros((STRIPE, D), jnp.float32)
    xp = jnp.pad(x, ((0, NP - N), (0, 0)))

    npad = EPAD - E
    pad_ids = jax.lax.iota(jnp.int32, npad)
    src_p = jnp.concatenate([src, pad_ids % N])
    dst_p = jnp.concatenate([dst, N + pad_ids % (NP - N)])

    deg2 = _deg_call(dst, ones_ch, zrow)
    dega = deg2[0].reshape(GRID, 1, RB)
    degb = deg2[1].reshape(GRID, 1, RB)
    y = _tc1_call(xp, W1, dega, degb)
    acc2 = _msg_call(y, src_p, dst_p, zblk)
    out = _tc2_call(acc2, y, xp, dega, degb,
                    b1.reshape(1, D), ln_gamma.reshape(1, D),
                    ln_beta.reshape(1, D), Wfc, bfc.reshape(1, C))
    return out[:N]


# speedup vs baseline: 2.6342x; 1.1698x over previous
import functools

import jax
import jax.numpy as jnp
from jax import lax
from jax.experimental import pallas as pl
from jax.experimental.pallas import tpu as pltpu
from jax.experimental.pallas import tpu_sc as plsc

N = 10000
E = 320000
D = 128
C = 64
DH = D // 2
NP = 10240
STRIPE = NP // 16
RB = 512
GRID = NP // RB

DEG_CH = 2000
DEG_PER_W = E // 32
MSG_CH = 160
MSG_NCH = 64
EPAD = 2 * 16 * MSG_NCH * MSG_CH


def _sc_mesh():
    return plsc.VectorSubcoreMesh(core_axis_name="c", subcore_axis_name="s")


def _deg_body(dst_hbm, ones_hbm, zrow_hbm, out_hbm, idx_v, ones_v, deg_s):
    c = lax.axis_index("c")
    s = lax.axis_index("s")
    pltpu.sync_copy(zrow_hbm, deg_s.at[pl.ds(s * STRIPE, STRIPE)])
    pltpu.sync_copy(ones_hbm, ones_v)
    plsc.subcore_barrier()
    base = (s * 2 + c) * DEG_PER_W
    for k in range(DEG_PER_W // DEG_CH):
        pltpu.sync_copy(dst_hbm.at[pl.ds(base + k * DEG_CH, DEG_CH)], idx_v)
        pltpu.sync_copy(ones_v, deg_s.at[idx_v], add=True)
    plsc.subcore_barrier()
    pltpu.sync_copy(deg_s.at[pl.ds(s * STRIPE, STRIPE)],
                    out_hbm.at[c, pl.ds(s * STRIPE, STRIPE)])


def _deg_call(dst, ones_ch, zrow):
    f = pl.kernel(
        _deg_body,
        out_type=jax.ShapeDtypeStruct((2, NP), jnp.float32),
        mesh=_sc_mesh(),
        scratch_types=[
            pltpu.VMEM((DEG_CH,), jnp.int32),
            pltpu.VMEM((DEG_CH,), jnp.float32),
            pltpu.VMEM_SHARED((NP,), jnp.float32),
        ],
    )
    return f(dst, ones_ch, zrow)


def _msg_body(y_hbm, src_hbm, dst_hbm, zblk_hbm, out_hbm,
              isrc0, idst0, isrc1, idst1, rows0, rows1, acc_s, sem0, sem1):
    c = lax.axis_index("c")
    s = lax.axis_index("s")
    pltpu.sync_copy(zblk_hbm, acc_s.at[pl.ds(s * STRIPE, STRIPE), :])
    plsc.subcore_barrier()
    ebase = (c * 16 + s) * (MSG_NCH * MSG_CH)

    def load_idx(k, isrc, idst):
        off = ebase + k * MSG_CH
        pltpu.sync_copy(src_hbm.at[pl.ds(off, MSG_CH)], isrc)
        pltpu.sync_copy(dst_hbm.at[pl.ds(off, MSG_CH)], idst)

    def issue(isrc, rows, sem):
        return pltpu.async_copy(y_hbm.at[isrc], rows, sem)

    def wait(isrc, rows, sem):
        pltpu.make_async_copy(y_hbm.at[isrc], rows, sem).wait()

    def scat(idst, rows):
        pltpu.sync_copy(rows, acc_s.at[idst], add=True)

    load_idx(0, isrc0, idst0)
    issue(isrc0, rows0, sem0)

    def pair(j, carry):
        a = 2 * j
        load_idx(a + 1, isrc1, idst1)
        issue(isrc1, rows1, sem1)
        wait(isrc0, rows0, sem0)
        scat(idst0, rows0)

        @pl.when(a + 2 < MSG_NCH)
        def _():
            load_idx(a + 2, isrc0, idst0)
            issue(isrc0, rows0, sem0)

        wait(isrc1, rows1, sem1)
        scat(idst1, rows1)
        return carry

    lax.fori_loop(0, MSG_NCH // 2, pair, 0)
    plsc.subcore_barrier()
    pltpu.sync_copy(acc_s.at[pl.ds(s * STRIPE, STRIPE), :],
                    out_hbm.at[c, pl.ds(s * STRIPE, STRIPE), :])


def _msg_call(y, src, dst, zblk):
    f = pl.kernel(
        _msg_body,
        out_type=jax.ShapeDtypeStruct((2, NP, D), jnp.float32),
        mesh=_sc_mesh(),
        scratch_types=[
            pltpu.VMEM((MSG_CH,), jnp.int32),
            pltpu.VMEM((MSG_CH,), jnp.int32),
            pltpu.VMEM((MSG_CH,), jnp.int32),
            pltpu.VMEM((MSG_CH,), jnp.int32),
            pltpu.VMEM((MSG_CH, D), jnp.float32),
            pltpu.VMEM((MSG_CH, D), jnp.float32),
            pltpu.VMEM_SHARED((NP, D), jnp.float32),
            pltpu.SemaphoreType.DMA,
            pltpu.SemaphoreType.DMA,
        ],
    )
    return f(y, src, dst, zblk)


def _tc1_body(x_ref, w_ref, dega_ref, degb_ref, y_ref):
    xw = jnp.dot(x_ref[...], w_ref[...], preferred_element_type=jnp.float32)
    deg = 1.0 + (dega_ref[...] + degb_ref[...]).reshape(RB)
    dinv = lax.rsqrt(deg)
    y_ref[...] = xw * dinv[:, None]


def _tc1_call(xp, W1, dega, degb):
    return pl.pallas_call(
        _tc1_body,
        grid=(GRID,),
        in_specs=[
            pl.BlockSpec((RB, D), lambda i: (i, 0)),
            pl.BlockSpec((D, D), lambda i: (0, 0)),
            pl.BlockSpec((1, 1, RB), lambda i: (i, 0, 0)),
            pl.BlockSpec((1, 1, RB), lambda i: (i, 0, 0)),
        ],
        out_specs=pl.BlockSpec((RB, D), lambda i: (i, 0)),
        out_shape=jax.ShapeDtypeStruct((NP, D), jnp.float32),
    )(xp, W1, dega, degb)


def _tc2_body(acc_ref, y_ref, x_ref, dega_ref, degb_ref, b1_ref, g_ref,
              be_ref, wfc_ref, bfc_ref, o_ref):
    acc = acc_ref[0] + acc_ref[1] + y_ref[...]
    deg = 1.0 + (dega_ref[...] + degb_ref[...]).reshape(RB)
    dinv = lax.rsqrt(deg)
    x = x_ref[...]
    h = b1_ref[...] + acc * dinv[:, None]
    h = jnp.maximum(h, 0.0) * x
    mean = jnp.mean(h, axis=-1, keepdims=True)
    var = jnp.mean((h - mean) ** 2, axis=-1, keepdims=True)
    h = (h - mean) * lax.rsqrt(var + 1e-5) * g_ref[...] + be_ref[...]
    h = h + x
    o_ref[...] = jnp.dot(h, wfc_ref[...],
                         preferred_element_type=jnp.float32) + bfc_ref[...]


def _tc2_call(acc2, y2, xp, dega, degb, b1, g, be, Wfc, bfc):
    return pl.pallas_call(
        _tc2_body,
        grid=(GRID,),
        in_specs=[
            pl.BlockSpec((2, RB, D), lambda i: (0, i, 0)),
            pl.BlockSpec((RB, D), lambda i: (i, 0)),
            pl.BlockSpec((RB, D), lambda i: (i, 0)),
            pl.BlockSpec((1, 1, RB), lambda i: (i, 0, 0)),
            pl.BlockSpec((1, 1, RB), lambda i: (i, 0, 0)),
            pl.BlockSpec((1, D), lambda i: (0, 0)),
            pl.BlockSpec((1, D), lambda i: (0, 0)),
            pl.BlockSpec((1, D), lambda i: (0, 0)),
            pl.BlockSpec((D, C), lambda i: (0, 0)),
            pl.BlockSpec((1, C), lambda i: (0, 0)),
        ],
        out_specs=pl.BlockSpec((RB, C), lambda i: (i, 0)),
        out_shape=jax.ShapeDtypeStruct((NP, C), jnp.float32),
    )(acc2, y2, xp, dega, degb, b1, g, be, Wfc, bfc)


def kernel(x, edge_index, W1, b1, ln_gamma, ln_beta, Wfc, bfc):
    src = edge_index[0]
    dst = edge_index[1]
    ones_ch = jnp.ones((DEG_CH,), jnp.float32)
    zrow = jnp.zeros((STRIPE,), jnp.float32)
    zblk = jnp.zeros((STRIPE, D), jnp.float32)
    xp = jnp.pad(x, ((0, NP - N), (0, 0)))

    npad = EPAD - E
    pad_ids = jax.lax.iota(jnp.int32, npad)
    src_p = jnp.concatenate([src, pad_ids % N])
    dst_p = jnp.concatenate([dst, N + pad_ids % (NP - N)])

    deg2 = _deg_call(dst, ones_ch, zrow)
    dega = deg2[0].reshape(GRID, 1, RB)
    degb = deg2[1].reshape(GRID, 1, RB)
    y = _tc1_call(xp, W1, dega, degb)
    acc2 = _msg_call(y, src_p, dst_p, zblk)
    out = _tc2_call(acc2, y, xp, dega, degb,
                    b1.reshape(1, D), ln_gamma.reshape(1, D),
                    ln_beta.reshape(1, D), Wfc, bfc.reshape(1, C))
    return out[:N]


# speedup vs baseline: 2.6496x; 1.0058x over previous
import functools

import jax
import jax.numpy as jnp
from jax import lax
from jax.experimental import pallas as pl
from jax.experimental.pallas import tpu as pltpu
from jax.experimental.pallas import tpu_sc as plsc

N = 10000
E = 320000
D = 128
C = 64
DH = D // 2
NP = 10240
STRIPE = NP // 16
RB = 512
GRID = NP // RB

DEG_CH = 2000
DEG_PER_W = E // 32
MSG_CH = 160
MSG_NCH = 64
EPAD = 2 * 16 * MSG_NCH * MSG_CH


def _sc_mesh():
    return plsc.VectorSubcoreMesh(core_axis_name="c", subcore_axis_name="s")


def _deg_body(dst_hbm, ones_hbm, zrow_hbm, out_hbm, idx_v, ones_v, deg_s):
    c = lax.axis_index("c")
    s = lax.axis_index("s")
    pltpu.sync_copy(zrow_hbm, deg_s.at[pl.ds(s * STRIPE, STRIPE)])
    pltpu.sync_copy(ones_hbm, ones_v)
    plsc.subcore_barrier()
    base = (s * 2 + c) * DEG_PER_W
    for k in range(DEG_PER_W // DEG_CH):
        pltpu.sync_copy(dst_hbm.at[pl.ds(base + k * DEG_CH, DEG_CH)], idx_v)
        pltpu.sync_copy(ones_v, deg_s.at[idx_v], add=True)
    plsc.subcore_barrier()
    pltpu.sync_copy(deg_s.at[pl.ds(s * STRIPE, STRIPE)],
                    out_hbm.at[c, pl.ds(s * STRIPE, STRIPE)])


def _deg_call(dst, ones_ch, zrow):
    f = pl.kernel(
        _deg_body,
        out_type=jax.ShapeDtypeStruct((2, NP), jnp.float32),
        mesh=_sc_mesh(),
        scratch_types=[
            pltpu.VMEM((DEG_CH,), jnp.int32),
            pltpu.VMEM((DEG_CH,), jnp.float32),
            pltpu.VMEM_SHARED((NP,), jnp.float32),
        ],
    )
    return f(dst, ones_ch, zrow)


def _msg_body(y_hbm, src_hbm, dst_hbm, zblk_hbm, out_hbm,
              isrc0, idst0, isrc1, idst1, rows0, rows1, acc_s,
              sem0, sem1, sscat0, sscat1):
    c = lax.axis_index("c")
    s = lax.axis_index("s")
    pltpu.sync_copy(zblk_hbm, acc_s.at[pl.ds(s * STRIPE, STRIPE), :])
    plsc.subcore_barrier()
    ebase = (c * 16 + s) * (MSG_NCH * MSG_CH)

    def load_idx(k, isrc, idst):
        off = ebase + k * MSG_CH
        pltpu.sync_copy(src_hbm.at[pl.ds(off, MSG_CH)], isrc)
        pltpu.sync_copy(dst_hbm.at[pl.ds(off, MSG_CH)], idst)

    def issue(isrc, rows, sem):
        pltpu.async_copy(y_hbm.at[isrc], rows, sem)

    def wait_g(isrc, rows, sem):
        pltpu.make_async_copy(y_hbm.at[isrc], rows, sem).wait()

    def scat(idst, rows, sem):
        pltpu.async_copy(rows, acc_s.at[idst], sem, add=True)

    def wait_s(idst, rows, sem):
        pltpu.make_async_copy(rows, acc_s.at[idst], sem).wait()

    load_idx(0, isrc0, idst0)
    issue(isrc0, rows0, sem0)
    load_idx(1, isrc1, idst1)
    issue(isrc1, rows1, sem1)

    def pair(j, carry):
        a = 2 * j
        wait_g(isrc0, rows0, sem0)
        scat(idst0, rows0, sscat0)
        wait_g(isrc1, rows1, sem1)
        scat(idst1, rows1, sscat1)
        wait_s(idst0, rows0, sscat0)

        @pl.when(a + 2 < MSG_NCH)
        def _():
            load_idx(a + 2, isrc0, idst0)
            issue(isrc0, rows0, sem0)

        wait_s(idst1, rows1, sscat1)

        @pl.when(a + 3 < MSG_NCH)
        def _():
            load_idx(a + 3, isrc1, idst1)
            issue(isrc1, rows1, sem1)

        return carry

    lax.fori_loop(0, MSG_NCH // 2, pair, 0)
    plsc.subcore_barrier()
    pltpu.sync_copy(acc_s.at[pl.ds(s * STRIPE, STRIPE), :],
                    out_hbm.at[c, pl.ds(s * STRIPE, STRIPE), :])


def _msg_call(y, src, dst, zblk):
    f = pl.kernel(
        _msg_body,
        out_type=jax.ShapeDtypeStruct((2, NP, D), jnp.float32),
        mesh=_sc_mesh(),
        scratch_types=[
            pltpu.VMEM((MSG_CH,), jnp.int32),
            pltpu.VMEM((MSG_CH,), jnp.int32),
            pltpu.VMEM((MSG_CH,), jnp.int32),
            pltpu.VMEM((MSG_CH,), jnp.int32),
            pltpu.VMEM((MSG_CH, D), jnp.float32),
            pltpu.VMEM((MSG_CH, D), jnp.float32),
            pltpu.VMEM_SHARED((NP, D), jnp.float32),
            pltpu.SemaphoreType.DMA,
            pltpu.SemaphoreType.DMA,
            pltpu.SemaphoreType.DMA,
            pltpu.SemaphoreType.DMA,
        ],
    )
    return f(y, src, dst, zblk)


def _tc1_body(x_ref, w_ref, dega_ref, degb_ref, y_ref):
    xw = jnp.dot(x_ref[...], w_ref[...], preferred_element_type=jnp.float32)
    deg = 1.0 + (dega_ref[...] + degb_ref[...]).reshape(RB)
    dinv = lax.rsqrt(deg)
    y_ref[...] = xw * dinv[:, None]


def _tc1_call(xp, W1, dega, degb):
    return pl.pallas_call(
        _tc1_body,
        grid=(GRID,),
        in_specs=[
            pl.BlockSpec((RB, D), lambda i: (i, 0)),
            pl.BlockSpec((D, D), lambda i: (0, 0)),
            pl.BlockSpec((1, 1, RB), lambda i: (i, 0, 0)),
            pl.BlockSpec((1, 1, RB), lambda i: (i, 0, 0)),
        ],
        out_specs=pl.BlockSpec((RB, D), lambda i: (i, 0)),
        out_shape=jax.ShapeDtypeStruct((NP, D), jnp.float32),
    )(xp, W1, dega, degb)


def _tc2_body(acc_ref, y_ref, x_ref, dega_ref, degb_ref, b1_ref, g_ref,
              be_ref, wfc_ref, bfc_ref, o_ref):
    acc = acc_ref[0] + acc_ref[1] + y_ref[...]
    deg = 1.0 + (dega_ref[...] + degb_ref[...]).reshape(RB)
    dinv = lax.rsqrt(deg)
    x = x_ref[...]
    h = b1_ref[...] + acc * dinv[:, None]
    h = jnp.maximum(h, 0.0) * x
    mean = jnp.mean(h, axis=-1, keepdims=True)
    var = jnp.mean((h - mean) ** 2, axis=-1, keepdims=True)
    h = (h - mean) * lax.rsqrt(var + 1e-5) * g_ref[...] + be_ref[...]
    h = h + x
    o_ref[...] = jnp.dot(h, wfc_ref[...],
                         preferred_element_type=jnp.float32) + bfc_ref[...]


def _tc2_call(acc2, y2, xp, dega, degb, b1, g, be, Wfc, bfc):
    return pl.pallas_call(
        _tc2_body,
        grid=(GRID,),
        in_specs=[
            pl.BlockSpec((2, RB, D), lambda i: (0, i, 0)),
            pl.BlockSpec((RB, D), lambda i: (i, 0)),
            pl.BlockSpec((RB, D), lambda i: (i, 0)),
            pl.BlockSpec((1, 1, RB), lambda i: (i, 0, 0)),
            pl.BlockSpec((1, 1, RB), lambda i: (i, 0, 0)),
            pl.BlockSpec((1, D), lambda i: (0, 0)),
            pl.BlockSpec((1, D), lambda i: (0, 0)),
            pl.BlockSpec((1, D), lambda i: (0, 0)),
            pl.BlockSpec((D, C), lambda i: (0, 0)),
            pl.BlockSpec((1, C), lambda i: (0, 0)),
        ],
        out_specs=pl.BlockSpec((RB, C), lambda i: (i, 0)),
        out_shape=jax.ShapeDtypeStruct((NP, C), jnp.float32),
    )(acc2, y2, xp, dega, degb, b1, g, be, Wfc, bfc)


def kernel(x, edge_index, W1, b1, ln_gamma, ln_beta, Wfc, bfc):
    src = edge_index[0]
    dst = edge_index[1]
    ones_ch = jnp.ones((DEG_CH,), jnp.float32)
    zrow = jnp.zeros((STRIPE,), jnp.float32)
    zblk = jnp.zeros((STRIPE, D), jnp.float32)
    xp = jnp.pad(x, ((0, NP - N), (0, 0)))

    npad = EPAD - E
    pad_ids = jax.lax.iota(jnp.int32, npad)
    src_p = jnp.concatenate([src, pad_ids % N])
    dst_p = jnp.concatenate([dst, N + pad_ids % (NP - N)])

    deg2 = _deg_call(dst, ones_ch, zrow)
    dega = deg2[0].reshape(GRID, 1, RB)
    degb = deg2[1].reshape(GRID, 1, RB)
    y = _tc1_call(xp, W1, dega, degb)
    acc2 = _msg_call(y, src_p, dst_p, zblk)
    out = _tc2_call(acc2, y, xp, dega, degb,
                    b1.reshape(1, D), ln_gamma.reshape(1, D),
                    ln_beta.reshape(1, D), Wfc, bfc.reshape(1, C))
    return out[:N]


# speedup vs baseline: 2.7477x; 1.0371x over previous
import functools

import jax
import jax.numpy as jnp
from jax import lax
from jax.experimental import pallas as pl
from jax.experimental.pallas import tpu as pltpu
from jax.experimental.pallas import tpu_sc as plsc

N = 10000
E = 320000
D = 128
C = 64
DH = D // 2
NP = 10240
STRIPE = NP // 16
RB = 512
GRID = NP // RB

DEG_CH = 10000
DEG_PER_W = E // 32
MSG_CH = 176
MSG_NCH = 58
EPAD = 2 * 16 * MSG_NCH * MSG_CH


def _sc_mesh():
    return plsc.VectorSubcoreMesh(core_axis_name="c", subcore_axis_name="s")


def _deg_body(dst_hbm, ones_hbm, zrow_hbm, out_hbm, idx_v, ones_v, deg_s):
    c = lax.axis_index("c")
    s = lax.axis_index("s")
    pltpu.sync_copy(zrow_hbm, deg_s.at[pl.ds(s * STRIPE, STRIPE)])
    pltpu.sync_copy(ones_hbm, ones_v)
    plsc.subcore_barrier()
    base = (s * 2 + c) * DEG_PER_W
    for k in range(DEG_PER_W // DEG_CH):
        pltpu.sync_copy(dst_hbm.at[pl.ds(base + k * DEG_CH, DEG_CH)], idx_v)
        pltpu.sync_copy(ones_v, deg_s.at[idx_v], add=True)
    plsc.subcore_barrier()
    pltpu.sync_copy(deg_s.at[pl.ds(s * STRIPE, STRIPE)],
                    out_hbm.at[c, pl.ds(s * STRIPE, STRIPE)])


def _deg_call(dst, ones_ch, zrow):
    f = pl.kernel(
        _deg_body,
        out_type=jax.ShapeDtypeStruct((2, NP), jnp.float32),
        mesh=_sc_mesh(),
        scratch_types=[
            pltpu.VMEM((DEG_CH,), jnp.int32),
            pltpu.VMEM((DEG_CH,), jnp.float32),
            pltpu.VMEM_SHARED((NP,), jnp.float32),
        ],
    )
    return f(dst, ones_ch, zrow)


def _msg_body(y_hbm, src_hbm, dst_hbm, zblk_hbm, out_hbm,
              isrc0, idst0, isrc1, idst1, rows0, rows1, acc_s,
              sem0, sem1, sscat0, sscat1):
    c = lax.axis_index("c")
    s = lax.axis_index("s")
    pltpu.sync_copy(zblk_hbm, acc_s.at[pl.ds(s * STRIPE, STRIPE), :])
    plsc.subcore_barrier()
    ebase = (c * 16 + s) * (MSG_NCH * MSG_CH)

    def load_idx(k, isrc, idst):
        off = ebase + k * MSG_CH
        pltpu.sync_copy(src_hbm.at[pl.ds(off, MSG_CH)], isrc)
        pltpu.sync_copy(dst_hbm.at[pl.ds(off, MSG_CH)], idst)

    def issue(isrc, rows, sem):
        pltpu.async_copy(y_hbm.at[isrc], rows, sem)

    def wait_g(isrc, rows, sem):
        pltpu.make_async_copy(y_hbm.at[isrc], rows, sem).wait()

    def scat(idst, rows, sem):
        pltpu.async_copy(rows, acc_s.at[idst], sem, add=True)

    def wait_s(idst, rows, sem):
        pltpu.make_async_copy(rows, acc_s.at[idst], sem).wait()

    load_idx(0, isrc0, idst0)
    issue(isrc0, rows0, sem0)
    load_idx(1, isrc1, idst1)
    issue(isrc1, rows1, sem1)

    def pair(j, carry):
        a = 2 * j
        wait_g(isrc0, rows0, sem0)
        scat(idst0, rows0, sscat0)
        wait_g(isrc1, rows1, sem1)
        scat(idst1, rows1, sscat1)
        wait_s(idst0, rows0, sscat0)

        @pl.when(a + 2 < MSG_NCH)
        def _():
            load_idx(a + 2, isrc0, idst0)
            issue(isrc0, rows0, sem0)

        wait_s(idst1, rows1, sscat1)

        @pl.when(a + 3 < MSG_NCH)
        def _():
            load_idx(a + 3, isrc1, idst1)
            issue(isrc1, rows1, sem1)

        return carry

    lax.fori_loop(0, MSG_NCH // 2, pair, 0)
    plsc.subcore_barrier()
    pltpu.sync_copy(acc_s.at[pl.ds(s * STRIPE, STRIPE), :],
                    out_hbm.at[c, pl.ds(s * STRIPE, STRIPE), :])


def _msg_call(y, src, dst, zblk):
    f = pl.kernel(
        _msg_body,
        out_type=jax.ShapeDtypeStruct((2, NP, D), jnp.float32),
        mesh=_sc_mesh(),
        scratch_types=[
            pltpu.VMEM((MSG_CH,), jnp.int32),
            pltpu.VMEM((MSG_CH,), jnp.int32),
            pltpu.VMEM((MSG_CH,), jnp.int32),
            pltpu.VMEM((MSG_CH,), jnp.int32),
            pltpu.VMEM((MSG_CH, D), jnp.float32),
            pltpu.VMEM((MSG_CH, D), jnp.float32),
            pltpu.VMEM_SHARED((NP, D), jnp.float32),
            pltpu.SemaphoreType.DMA,
            pltpu.SemaphoreType.DMA,
            pltpu.SemaphoreType.DMA,
            pltpu.SemaphoreType.DMA,
        ],
    )
    return f(y, src, dst, zblk)


def _tc1_body(x_ref, w_ref, dega_ref, degb_ref, y_ref):
    xw = jnp.dot(x_ref[...], w_ref[...], preferred_element_type=jnp.float32)
    deg = 1.0 + (dega_ref[...] + degb_ref[...]).reshape(RB)
    dinv = lax.rsqrt(deg)
    y_ref[...] = xw * dinv[:, None]


def _tc1_call(xp, W1, dega, degb):
    return pl.pallas_call(
        _tc1_body,
        grid=(GRID,),
        in_specs=[
            pl.BlockSpec((RB, D), lambda i: (i, 0)),
            pl.BlockSpec((D, D), lambda i: (0, 0)),
            pl.BlockSpec((1, 1, RB), lambda i: (i, 0, 0)),
            pl.BlockSpec((1, 1, RB), lambda i: (i, 0, 0)),
        ],
        out_specs=pl.BlockSpec((RB, D), lambda i: (i, 0)),
        out_shape=jax.ShapeDtypeStruct((NP, D), jnp.float32),
    )(xp, W1, dega, degb)


def _tc2_body(acc_ref, y_ref, x_ref, dega_ref, degb_ref, b1_ref, g_ref,
              be_ref, wfc_ref, bfc_ref, o_ref):
    acc = acc_ref[0] + acc_ref[1] + y_ref[...]
    deg = 1.0 + (dega_ref[...] + degb_ref[...]).reshape(RB)
    dinv = lax.rsqrt(deg)
    x = x_ref[...]
    h = b1_ref[...] + acc * dinv[:, None]
    h = jnp.maximum(h, 0.0) * x
    mean = jnp.mean(h, axis=-1, keepdims=True)
    var = jnp.mean((h - mean) ** 2, axis=-1, keepdims=True)
    h = (h - mean) * lax.rsqrt(var + 1e-5) * g_ref[...] + be_ref[...]
    h = h + x
    o_ref[...] = jnp.dot(h, wfc_ref[...],
                         preferred_element_type=jnp.float32) + bfc_ref[...]


def _tc2_call(acc2, y2, xp, dega, degb, b1, g, be, Wfc, bfc):
    return pl.pallas_call(
        _tc2_body,
        grid=(GRID,),
        in_specs=[
            pl.BlockSpec((2, RB, D), lambda i: (0, i, 0)),
            pl.BlockSpec((RB, D), lambda i: (i, 0)),
            pl.BlockSpec((RB, D), lambda i: (i, 0)),
            pl.BlockSpec((1, 1, RB), lambda i: (i, 0, 0)),
            pl.BlockSpec((1, 1, RB), lambda i: (i, 0, 0)),
            pl.BlockSpec((1, D), lambda i: (0, 0)),
            pl.BlockSpec((1, D), lambda i: (0, 0)),
            pl.BlockSpec((1, D), lambda i: (0, 0)),
            pl.BlockSpec((D, C), lambda i: (0, 0)),
            pl.BlockSpec((1, C), lambda i: (0, 0)),
        ],
        out_specs=pl.BlockSpec((RB, C), lambda i: (i, 0)),
        out_shape=jax.ShapeDtypeStruct((NP, C), jnp.float32),
    )(acc2, y2, xp, dega, degb, b1, g, be, Wfc, bfc)


def kernel(x, edge_index, W1, b1, ln_gamma, ln_beta, Wfc, bfc):
    src = edge_index[0]
    dst = edge_index[1]
    ones_ch = jnp.ones((DEG_CH,), jnp.float32)
    zrow = jnp.zeros((STRIPE,), jnp.float32)
    zblk = jnp.zeros((STRIPE, D), jnp.float32)
    xp = jnp.pad(x, ((0, NP - N), (0, 0)))

    npad = EPAD - E
    pad_ids = jax.lax.iota(jnp.int32, npad)
    src_p = jnp.concatenate([src, pad_ids % N])
    dst_p = jnp.concatenate([dst, N + pad_ids % (NP - N)])

    deg2 = _deg_call(dst, ones_ch, zrow)
    dega = deg2[0].reshape(GRID, 1, RB)
    degb = deg2[1].reshape(GRID, 1, RB)
    y = _tc1_call(xp, W1, dega, degb)
    acc2 = _msg_call(y, src_p, dst_p, zblk)
    out = _tc2_call(acc2, y, xp, dega, degb,
                    b1.reshape(1, D), ln_gamma.reshape(1, D),
                    ln_beta.reshape(1, D), Wfc, bfc.reshape(1, C))
    return out[:N]


# speedup vs baseline: 2.7772x; 1.0107x over previous
import functools

import jax
import jax.numpy as jnp
from jax import lax
from jax.experimental import pallas as pl
from jax.experimental.pallas import tpu as pltpu
from jax.experimental.pallas import tpu_sc as plsc

N = 10000
E = 320000
D = 128
C = 64
DH = D // 2
NP = 10240
STRIPE = NP // 16
RB = 512
GRID = NP // RB

DEG_CH = 10000
DEG_PER_W = E // 32
MSG_CH = 176
MSG_NCH = 58
EPAD = 2 * 16 * MSG_NCH * MSG_CH


def _sc_mesh():
    return plsc.VectorSubcoreMesh(core_axis_name="c", subcore_axis_name="s")


def _deg_body(dst_hbm, ones_hbm, zrow_hbm, out_hbm, idx_v, ones_v, deg_s):
    c = lax.axis_index("c")
    s = lax.axis_index("s")
    pltpu.sync_copy(zrow_hbm, deg_s.at[pl.ds(s * STRIPE, STRIPE)])
    pltpu.sync_copy(ones_hbm, ones_v)
    plsc.subcore_barrier()
    base = (s * 2 + c) * DEG_PER_W
    for k in range(DEG_PER_W // DEG_CH):
        pltpu.sync_copy(dst_hbm.at[pl.ds(base + k * DEG_CH, DEG_CH)], idx_v)
        pltpu.sync_copy(ones_v, deg_s.at[idx_v], add=True)
    plsc.subcore_barrier()
    pltpu.sync_copy(deg_s.at[pl.ds(s * STRIPE, STRIPE)],
                    out_hbm.at[c, pl.ds(s * STRIPE, STRIPE)])


def _deg_call(dst, ones_ch, zrow):
    f = pl.kernel(
        _deg_body,
        out_type=jax.ShapeDtypeStruct((2, NP), jnp.float32),
        mesh=_sc_mesh(),
        scratch_types=[
            pltpu.VMEM((DEG_CH,), jnp.int32),
            pltpu.VMEM((DEG_CH,), jnp.float32),
            pltpu.VMEM_SHARED((NP,), jnp.float32),
        ],
    )
    return f(dst, ones_ch, zrow)


def _msg_body(y_hbm, src_hbm, dst_hbm, zblk_hbm, out_hbm,
              isrc0, idst0, isrc1, idst1, rows0, rows1, acc_s,
              sem0, sem1, sscat0, sscat1):
    c = lax.axis_index("c")
    s = lax.axis_index("s")
    pltpu.sync_copy(zblk_hbm, acc_s.at[pl.ds(s * STRIPE, STRIPE), :])
    plsc.subcore_barrier()
    ebase = (c * 16 + s) * (MSG_NCH * MSG_CH)

    def load_idx(k, isrc, idst):
        off = ebase + k * MSG_CH
        pltpu.sync_copy(src_hbm.at[pl.ds(off, MSG_CH)], isrc)
        pltpu.sync_copy(dst_hbm.at[pl.ds(off, MSG_CH)], idst)

    def issue(isrc, rows, sem):
        pltpu.async_copy(y_hbm.at[isrc], rows, sem)

    def wait_g(isrc, rows, sem):
        pltpu.make_async_copy(y_hbm.at[isrc], rows, sem).wait()

    def scat(idst, rows, sem):
        pltpu.async_copy(rows, acc_s.at[idst], sem, add=True)

    def wait_s(idst, rows, sem):
        pltpu.make_async_copy(rows, acc_s.at[idst], sem).wait()

    load_idx(0, isrc0, idst0)
    issue(isrc0, rows0, sem0)
    load_idx(1, isrc1, idst1)
    issue(isrc1, rows1, sem1)

    def pair(j, carry):
        a = 2 * j
        wait_g(isrc0, rows0, sem0)
        scat(idst0, rows0, sscat0)
        wait_g(isrc1, rows1, sem1)
        scat(idst1, rows1, sscat1)
        wait_s(idst0, rows0, sscat0)

        @pl.when(a + 2 < MSG_NCH)
        def _():
            load_idx(a + 2, isrc0, idst0)
            issue(isrc0, rows0, sem0)

        wait_s(idst1, rows1, sscat1)

        @pl.when(a + 3 < MSG_NCH)
        def _():
            load_idx(a + 3, isrc1, idst1)
            issue(isrc1, rows1, sem1)

        return carry

    lax.fori_loop(0, MSG_NCH // 2, pair, 0)
    plsc.subcore_barrier()
    pltpu.sync_copy(acc_s.at[pl.ds(s * STRIPE, STRIPE), :],
                    out_hbm.at[c, pl.ds(s * STRIPE, STRIPE), :])


def _msg_call(y, src, dst, zblk):
    f = pl.kernel(
        _msg_body,
        out_type=jax.ShapeDtypeStruct((2, NP, D), jnp.float32),
        mesh=_sc_mesh(),
        scratch_types=[
            pltpu.VMEM((MSG_CH,), jnp.int32),
            pltpu.VMEM((MSG_CH,), jnp.int32),
            pltpu.VMEM((MSG_CH,), jnp.int32),
            pltpu.VMEM((MSG_CH,), jnp.int32),
            pltpu.VMEM((MSG_CH, D), jnp.float32),
            pltpu.VMEM((MSG_CH, D), jnp.float32),
            pltpu.VMEM_SHARED((NP, D), jnp.float32),
            pltpu.SemaphoreType.DMA,
            pltpu.SemaphoreType.DMA,
            pltpu.SemaphoreType.DMA,
            pltpu.SemaphoreType.DMA,
        ],
    )
    return f(y, src, dst, zblk)


def _tc1_body(x_ref, w_ref, dega_ref, degb_ref, y_ref):
    xw = jnp.dot(x_ref[...], w_ref[...], preferred_element_type=jnp.float32)
    deg = 1.0 + (dega_ref[...] + degb_ref[...]).reshape(RB)
    dinv = lax.rsqrt(deg)
    y_ref[...] = xw * dinv[:, None]


def _tc1_call(xp, W1, dega, degb):
    return pl.pallas_call(
        _tc1_body,
        grid=(GRID,),
        in_specs=[
            pl.BlockSpec((RB, D), lambda i: (i, 0)),
            pl.BlockSpec((D, D), lambda i: (0, 0)),
            pl.BlockSpec((1, 1, RB), lambda i: (i, 0, 0)),
            pl.BlockSpec((1, 1, RB), lambda i: (i, 0, 0)),
        ],
        out_specs=pl.BlockSpec((RB, D), lambda i: (i, 0)),
        out_shape=jax.ShapeDtypeStruct((NP, D), jnp.float32),
    )(xp, W1, dega, degb)


def _tc2_body(acc_ref, y_ref, x_ref, dega_ref, degb_ref, b1_ref, g_ref,
              be_ref, wfc_ref, bfc_ref, o_ref):
    acc = acc_ref[0] + acc_ref[1] + y_ref[...]
    deg = 1.0 + (dega_ref[...] + degb_ref[...]).reshape(RB)
    dinv = lax.rsqrt(deg)
    x = x_ref[...]
    h = b1_ref[...] + acc * dinv[:, None]
    h = jnp.maximum(h, 0.0) * x
    mean = jnp.mean(h, axis=-1, keepdims=True)
    var = jnp.mean((h - mean) ** 2, axis=-1, keepdims=True)
    h = (h - mean) * lax.rsqrt(var + 1e-5) * g_ref[...] + be_ref[...]
    h = h + x
    o_ref[...] = jnp.dot(h, wfc_ref[...],
                         preferred_element_type=jnp.float32) + bfc_ref[...]


def _tc2_call(acc2, y2, xp, dega, degb, b1, g, be, Wfc, bfc):
    return pl.pallas_call(
        _tc2_body,
        grid=(GRID,),
        in_specs=[
            pl.BlockSpec((2, RB, D), lambda i: (0, i, 0)),
            pl.BlockSpec((RB, D), lambda i: (i, 0)),
            pl.BlockSpec((RB, D), lambda i: (i, 0)),
            pl.BlockSpec((1, 1, RB), lambda i: (i, 0, 0)),
            pl.BlockSpec((1, 1, RB), lambda i: (i, 0, 0)),
            pl.BlockSpec((1, D), lambda i: (0, 0)),
            pl.BlockSpec((1, D), lambda i: (0, 0)),
            pl.BlockSpec((1, D), lambda i: (0, 0)),
            pl.BlockSpec((D, C), lambda i: (0, 0)),
            pl.BlockSpec((1, C), lambda i: (0, 0)),
        ],
        out_specs=pl.BlockSpec((RB, C), lambda i: (i, 0)),
        out_shape=jax.ShapeDtypeStruct((N, C), jnp.float32),
    )(acc2, y2, xp, dega, degb, b1, g, be, Wfc, bfc)


def kernel(x, edge_index, W1, b1, ln_gamma, ln_beta, Wfc, bfc):
    src = edge_index[0]
    dst = edge_index[1]
    ones_ch = jnp.ones((DEG_CH,), jnp.float32)
    zrow = jnp.zeros((STRIPE,), jnp.float32)
    zblk = jnp.zeros((STRIPE, D), jnp.float32)

    npad = EPAD - E
    pad_ids = jax.lax.iota(jnp.int32, npad)
    src_p = jnp.concatenate([src, pad_ids % N])
    dst_p = jnp.concatenate([dst, N + pad_ids % (NP - N)])

    deg2 = _deg_call(dst, ones_ch, zrow)
    dega = deg2[0].reshape(GRID, 1, RB)
    degb = deg2[1].reshape(GRID, 1, RB)
    y = _tc1_call(x, W1, dega, degb)
    acc2 = _msg_call(y, src_p, dst_p, zblk)
    out = _tc2_call(acc2, y, x, dega, degb,
                    b1.reshape(1, D), ln_gamma.reshape(1, D),
                    ln_beta.reshape(1, D), Wfc, bfc.reshape(1, C))
    return out


# speedup vs baseline: 2.7779x; 1.0003x over previous
import functools

import jax
import jax.numpy as jnp
from jax import lax
from jax.experimental import pallas as pl
from jax.experimental.pallas import tpu as pltpu
from jax.experimental.pallas import tpu_sc as plsc

N = 10000
E = 320000
D = 128
C = 64
DH = D // 2
NP = 10240
STRIPE = NP // 16
RB = 512
GRID = NP // RB

DEG_CH = 10000
DEG_PER_W = E // 32
MSG_CH = 176
MSG_NCH = 58
EPAD = 2 * 16 * MSG_NCH * MSG_CH


def _sc_mesh():
    return plsc.VectorSubcoreMesh(core_axis_name="c", subcore_axis_name="s")


def _deg_body(dst_hbm, ones_hbm, zrow_hbm, out_hbm, idx_v, ones_v, deg_s):
    c = lax.axis_index("c")
    s = lax.axis_index("s")
    pltpu.sync_copy(zrow_hbm, deg_s.at[pl.ds(s * STRIPE, STRIPE)])
    pltpu.sync_copy(ones_hbm, ones_v)
    plsc.subcore_barrier()
    base = (s * 2 + c) * DEG_PER_W
    for k in range(DEG_PER_W // DEG_CH):
        pltpu.sync_copy(dst_hbm.at[pl.ds(base + k * DEG_CH, DEG_CH)], idx_v)
        pltpu.sync_copy(ones_v, deg_s.at[idx_v], add=True)
    plsc.subcore_barrier()
    pltpu.sync_copy(deg_s.at[pl.ds(s * STRIPE, STRIPE)],
                    out_hbm.at[c, pl.ds(s * STRIPE, STRIPE)])


def _deg_call(dst, ones_ch, zrow):
    f = pl.kernel(
        _deg_body,
        out_type=jax.ShapeDtypeStruct((2, NP), jnp.float32),
        mesh=_sc_mesh(),
        scratch_types=[
            pltpu.VMEM((DEG_CH,), jnp.int32),
            pltpu.VMEM((DEG_CH,), jnp.float32),
            pltpu.VMEM_SHARED((NP,), jnp.float32),
        ],
    )
    return f(dst, ones_ch, zrow)


def _msg_body(y_hbm, src_hbm, dst_hbm, zblk_hbm, out_hbm,
              isrc0, idst0, isrc1, idst1, rows0, rows1, acc_s,
              sem0, sem1, sscat0, sscat1, semz):
    c = lax.axis_index("c")
    s = lax.axis_index("s")
    zcp = pltpu.async_copy(zblk_hbm, acc_s.at[pl.ds(s * STRIPE, STRIPE), :],
                           semz)
    ebase = (c * 16 + s) * (MSG_NCH * MSG_CH)

    def load_idx(k, isrc, idst):
        off = ebase + k * MSG_CH
        pltpu.sync_copy(src_hbm.at[pl.ds(off, MSG_CH)], isrc)
        pltpu.sync_copy(dst_hbm.at[pl.ds(off, MSG_CH)], idst)

    def issue(isrc, rows, sem):
        pltpu.async_copy(y_hbm.at[isrc], rows, sem)

    def wait_g(isrc, rows, sem):
        pltpu.make_async_copy(y_hbm.at[isrc], rows, sem).wait()

    def scat(idst, rows, sem):
        pltpu.async_copy(rows, acc_s.at[idst], sem, add=True)

    def wait_s(idst, rows, sem):
        pltpu.make_async_copy(rows, acc_s.at[idst], sem).wait()

    load_idx(0, isrc0, idst0)
    issue(isrc0, rows0, sem0)
    load_idx(1, isrc1, idst1)
    issue(isrc1, rows1, sem1)
    zcp.wait()
    plsc.subcore_barrier()

    def pair(j, carry):
        a = 2 * j
        wait_g(isrc0, rows0, sem0)
        scat(idst0, rows0, sscat0)
        wait_g(isrc1, rows1, sem1)
        scat(idst1, rows1, sscat1)
        wait_s(idst0, rows0, sscat0)

        @pl.when(a + 2 < MSG_NCH)
        def _():
            load_idx(a + 2, isrc0, idst0)
            issue(isrc0, rows0, sem0)

        wait_s(idst1, rows1, sscat1)

        @pl.when(a + 3 < MSG_NCH)
        def _():
            load_idx(a + 3, isrc1, idst1)
            issue(isrc1, rows1, sem1)

        return carry

    lax.fori_loop(0, MSG_NCH // 2, pair, 0)
    plsc.subcore_barrier()
    pltpu.sync_copy(acc_s.at[pl.ds(s * STRIPE, STRIPE), :],
                    out_hbm.at[c, pl.ds(s * STRIPE, STRIPE), :])


def _msg_call(y, src, dst, zblk):
    f = pl.kernel(
        _msg_body,
        out_type=jax.ShapeDtypeStruct((2, NP, D), jnp.float32),
        mesh=_sc_mesh(),
        scratch_types=[
            pltpu.VMEM((MSG_CH,), jnp.int32),
            pltpu.VMEM((MSG_CH,), jnp.int32),
            pltpu.VMEM((MSG_CH,), jnp.int32),
            pltpu.VMEM((MSG_CH,), jnp.int32),
            pltpu.VMEM((MSG_CH, D), jnp.float32),
            pltpu.VMEM((MSG_CH, D), jnp.float32),
            pltpu.VMEM_SHARED((NP, D), jnp.float32),
            pltpu.SemaphoreType.DMA,
            pltpu.SemaphoreType.DMA,
            pltpu.SemaphoreType.DMA,
            pltpu.SemaphoreType.DMA,
            pltpu.SemaphoreType.DMA,
        ],
    )
    return f(y, src, dst, zblk)


def _tc1_body(x_ref, w_ref, dega_ref, degb_ref, y_ref):
    xw = jnp.dot(x_ref[...], w_ref[...], preferred_element_type=jnp.float32)
    deg = 1.0 + (dega_ref[...] + degb_ref[...]).reshape(RB)
    dinv = lax.rsqrt(deg)
    y_ref[...] = xw * dinv[:, None]


def _tc1_call(xp, W1, dega, degb):
    return pl.pallas_call(
        _tc1_body,
        grid=(GRID,),
        in_specs=[
            pl.BlockSpec((RB, D), lambda i: (i, 0)),
            pl.BlockSpec((D, D), lambda i: (0, 0)),
            pl.BlockSpec((1, 1, RB), lambda i: (i, 0, 0)),
            pl.BlockSpec((1, 1, RB), lambda i: (i, 0, 0)),
        ],
        out_specs=pl.BlockSpec((RB, D), lambda i: (i, 0)),
        out_shape=jax.ShapeDtypeStruct((NP, D), jnp.float32),
    )(xp, W1, dega, degb)


def _tc2_body(acc_ref, y_ref, x_ref, dega_ref, degb_ref, b1_ref, g_ref,
              be_ref, wfc_ref, bfc_ref, o_ref):
    acc = acc_ref[0] + acc_ref[1] + y_ref[...]
    deg = 1.0 + (dega_ref[...] + degb_ref[...]).reshape(RB)
    dinv = lax.rsqrt(deg)
    x = x_ref[...]
    h = b1_ref[...] + acc * dinv[:, None]
    h = jnp.maximum(h, 0.0) * x
    mean = jnp.mean(h, axis=-1, keepdims=True)
    var = jnp.mean((h - mean) ** 2, axis=-1, keepdims=True)
    h = (h - mean) * lax.rsqrt(var + 1e-5) * g_ref[...] + be_ref[...]
    h = h + x
    o_ref[...] = jnp.dot(h, wfc_ref[...],
                         preferred_element_type=jnp.float32) + bfc_ref[...]


def _tc2_call(acc2, y2, xp, dega, degb, b1, g, be, Wfc, bfc):
    return pl.pallas_call(
        _tc2_body,
        grid=(GRID,),
        in_specs=[
            pl.BlockSpec((2, RB, D), lambda i: (0, i, 0)),
            pl.BlockSpec((RB, D), lambda i: (i, 0)),
            pl.BlockSpec((RB, D), lambda i: (i, 0)),
            pl.BlockSpec((1, 1, RB), lambda i: (i, 0, 0)),
            pl.BlockSpec((1, 1, RB), lambda i: (i, 0, 0)),
            pl.BlockSpec((1, D), lambda i: (0, 0)),
            pl.BlockSpec((1, D), lambda i: (0, 0)),
            pl.BlockSpec((1, D), lambda i: (0, 0)),
            pl.BlockSpec((D, C), lambda i: (0, 0)),
            pl.BlockSpec((1, C), lambda i: (0, 0)),
        ],
        out_specs=pl.BlockSpec((RB, C), lambda i: (i, 0)),
        out_shape=jax.ShapeDtypeStruct((N, C), jnp.float32),
    )(acc2, y2, xp, dega, degb, b1, g, be, Wfc, bfc)


def kernel(x, edge_index, W1, b1, ln_gamma, ln_beta, Wfc, bfc):
    src = edge_index[0]
    dst = edge_index[1]
    ones_ch = jnp.ones((DEG_CH,), jnp.float32)
    zrow = jnp.zeros((STRIPE,), jnp.float32)
    zblk = jnp.zeros((STRIPE, D), jnp.float32)

    npad = EPAD - E
    pad_ids = jax.lax.iota(jnp.int32, npad)
    src_p = jnp.concatenate([src, pad_ids % N])
    dst_p = jnp.concatenate([dst, N + pad_ids % (NP - N)])

    deg2 = _deg_call(dst, ones_ch, zrow)
    dega = deg2[0].reshape(GRID, 1, RB)
    degb = deg2[1].reshape(GRID, 1, RB)
    y = _tc1_call(x, W1, dega, degb)
    acc2 = _msg_call(y, src_p, dst_p, zblk)
    out = _tc2_call(acc2, y, x, dega, degb,
                    b1.reshape(1, D), ln_gamma.reshape(1, D),
                    ln_beta.reshape(1, D), Wfc, bfc.reshape(1, C))
    return out


# speedup vs baseline: 2.7819x; 1.0014x over previous
import jax
import jax.numpy as jnp
from jax import lax
from jax.experimental import pallas as pl
from jax.experimental.pallas import tpu as pltpu
from jax.experimental.pallas import tpu_sc as plsc

N = 10000
E = 320000
D = 128
C = 64
NP = 10240
STRIPE = NP // 16
RB = 512
GRID = NP // RB

DEG_CH = 10000
DEG_PER_W = E // 32
MSG_CH = 176
MSG_NCH = 58
EPAD = 2 * 16 * MSG_NCH * MSG_CH


def _sc_mesh():
    return plsc.VectorSubcoreMesh(core_axis_name="c", subcore_axis_name="s")


def _deg_body(dst_hbm, ones_hbm, zrow_hbm, out_hbm, idx_v, ones_v, deg_s):
    c = lax.axis_index("c")
    s = lax.axis_index("s")
    pltpu.sync_copy(zrow_hbm, deg_s.at[pl.ds(s * STRIPE, STRIPE)])
    pltpu.sync_copy(ones_hbm, ones_v)
    plsc.subcore_barrier()
    base = (s * 2 + c) * DEG_PER_W
    for k in range(DEG_PER_W // DEG_CH):
        pltpu.sync_copy(dst_hbm.at[pl.ds(base + k * DEG_CH, DEG_CH)], idx_v)
        pltpu.sync_copy(ones_v, deg_s.at[idx_v], add=True)
    plsc.subcore_barrier()
    pltpu.sync_copy(deg_s.at[pl.ds(s * STRIPE, STRIPE)],
                    out_hbm.at[c, pl.ds(s * STRIPE, STRIPE)])


def _deg_call(dst, ones_ch, zrow):
    f = pl.kernel(
        _deg_body,
        out_type=jax.ShapeDtypeStruct((2, NP), jnp.float32),
        mesh=_sc_mesh(),
        scratch_types=[
            pltpu.VMEM((DEG_CH,), jnp.int32),
            pltpu.VMEM((DEG_CH,), jnp.float32),
            pltpu.VMEM_SHARED((NP,), jnp.float32),
        ],
    )
    return f(dst, ones_ch, zrow)


def _msg_body(y_hbm, src_hbm, dst_hbm, zblk_hbm, out_hbm,
              isrc0, idst0, isrc1, idst1, rows0, rows1, acc_s,
              sem0, sem1, sscat0, sscat1, semz):
    c = lax.axis_index("c")
    s = lax.axis_index("s")
    zcp = pltpu.async_copy(zblk_hbm, acc_s.at[pl.ds(s * STRIPE, STRIPE), :],
                           semz)
    ebase = (c * 16 + s) * (MSG_NCH * MSG_CH)

    def load_idx(k, isrc, idst):
        off = ebase + k * MSG_CH
        pltpu.sync_copy(src_hbm.at[pl.ds(off, MSG_CH)], isrc)
        pltpu.sync_copy(dst_hbm.at[pl.ds(off, MSG_CH)], idst)

    def issue(isrc, rows, sem):
        pltpu.async_copy(y_hbm.at[isrc], rows, sem)

    def wait_g(isrc, rows, sem):
        pltpu.make_async_copy(y_hbm.at[isrc], rows, sem).wait()

    def scat(idst, rows, sem):
        pltpu.async_copy(rows, acc_s.at[idst], sem, add=True)

    def wait_s(idst, rows, sem):
        pltpu.make_async_copy(rows, acc_s.at[idst], sem).wait()

    load_idx(0, isrc0, idst0)
    issue(isrc0, rows0, sem0)
    load_idx(1, isrc1, idst1)
    issue(isrc1, rows1, sem1)
    zcp.wait()
    plsc.subcore_barrier()

    def pair(j, carry):
        a = 2 * j
        wait_g(isrc0, rows0, sem0)
        scat(idst0, rows0, sscat0)
        wait_g(isrc1, rows1, sem1)
        scat(idst1, rows1, sscat1)
        wait_s(idst0, rows0, sscat0)

        @pl.when(a + 2 < MSG_NCH)
        def _():
            load_idx(a + 2, isrc0, idst0)
            issue(isrc0, rows0, sem0)

        wait_s(idst1, rows1, sscat1)

        @pl.when(a + 3 < MSG_NCH)
        def _():
            load_idx(a + 3, isrc1, idst1)
            issue(isrc1, rows1, sem1)

        return carry

    lax.fori_loop(0, MSG_NCH // 2, pair, 0)
    plsc.subcore_barrier()
    pltpu.sync_copy(acc_s.at[pl.ds(s * STRIPE, STRIPE), :],
                    out_hbm.at[c, pl.ds(s * STRIPE, STRIPE), :])


def _msg_call(y, src, dst, zblk):
    f = pl.kernel(
        _msg_body,
        out_type=jax.ShapeDtypeStruct((2, NP, D), jnp.float32),
        mesh=_sc_mesh(),
        scratch_types=[
            pltpu.VMEM((MSG_CH,), jnp.int32),
            pltpu.VMEM((MSG_CH,), jnp.int32),
            pltpu.VMEM((MSG_CH,), jnp.int32),
            pltpu.VMEM((MSG_CH,), jnp.int32),
            pltpu.VMEM((MSG_CH, D), jnp.float32),
            pltpu.VMEM((MSG_CH, D), jnp.float32),
            pltpu.VMEM_SHARED((NP, D), jnp.float32),
            pltpu.SemaphoreType.DMA,
            pltpu.SemaphoreType.DMA,
            pltpu.SemaphoreType.DMA,
            pltpu.SemaphoreType.DMA,
            pltpu.SemaphoreType.DMA,
        ],
    )
    return f(y, src, dst, zblk)


def _tc1_body(x_ref, w_ref, dega_ref, degb_ref, y_ref):
    xw = jnp.dot(x_ref[...], w_ref[...], preferred_element_type=jnp.float32)
    deg = 1.0 + (dega_ref[...] + degb_ref[...]).reshape(RB)
    dinv = lax.rsqrt(deg)
    y_ref[...] = xw * dinv[:, None]


def _tc1_call(xp, W1, dega, degb):
    return pl.pallas_call(
        _tc1_body,
        grid=(GRID,),
        in_specs=[
            pl.BlockSpec((RB, D), lambda i: (i, 0)),
            pl.BlockSpec((D, D), lambda i: (0, 0)),
            pl.BlockSpec((1, 1, RB), lambda i: (i, 0, 0)),
            pl.BlockSpec((1, 1, RB), lambda i: (i, 0, 0)),
        ],
        out_specs=pl.BlockSpec((RB, D), lambda i: (i, 0)),
        out_shape=jax.ShapeDtypeStruct((NP, D), jnp.float32),
    )(xp, W1, dega, degb)


def _tc2_body(acc_ref, y_ref, x_ref, dega_ref, degb_ref, b1_ref, g_ref,
              be_ref, wfc_ref, bfc_ref, o_ref):
    acc = acc_ref[0] + acc_ref[1] + y_ref[...]
    deg = 1.0 + (dega_ref[...] + degb_ref[...]).reshape(RB)
    dinv = lax.rsqrt(deg)
    x = x_ref[...]
    h = b1_ref[...] + acc * dinv[:, None]
    h = jnp.maximum(h, 0.0) * x
    mean = jnp.mean(h, axis=-1, keepdims=True)
    var = jnp.mean((h - mean) ** 2, axis=-1, keepdims=True)
    h = (h - mean) * lax.rsqrt(var + 1e-5) * g_ref[...] + be_ref[...]
    h = h + x
    o_ref[...] = jnp.dot(h, wfc_ref[...],
                         preferred_element_type=jnp.float32) + bfc_ref[...]


def _tc2_call(acc2, y2, xp, dega, degb, b1, g, be, Wfc, bfc):
    return pl.pallas_call(
        _tc2_body,
        grid=(GRID,),
        in_specs=[
            pl.BlockSpec((2, RB, D), lambda i: (0, i, 0)),
            pl.BlockSpec((RB, D), lambda i: (i, 0)),
            pl.BlockSpec((RB, D), lambda i: (i, 0)),
            pl.BlockSpec((1, 1, RB), lambda i: (i, 0, 0)),
            pl.BlockSpec((1, 1, RB), lambda i: (i, 0, 0)),
            pl.BlockSpec((1, D), lambda i: (0, 0)),
            pl.BlockSpec((1, D), lambda i: (0, 0)),
            pl.BlockSpec((1, D), lambda i: (0, 0)),
            pl.BlockSpec((D, C), lambda i: (0, 0)),
            pl.BlockSpec((1, C), lambda i: (0, 0)),
        ],
        out_specs=pl.BlockSpec((RB, C), lambda i: (i, 0)),
        out_shape=jax.ShapeDtypeStruct((N, C), jnp.float32),
    )(acc2, y2, xp, dega, degb, b1, g, be, Wfc, bfc)


def kernel(x, edge_index, W1, b1, ln_gamma, ln_beta, Wfc, bfc):
    src = edge_index[0]
    dst = edge_index[1]
    ones_ch = jnp.ones((DEG_CH,), jnp.float32)
    zrow = jnp.zeros((STRIPE,), jnp.float32)
    zblk = jnp.zeros((STRIPE, D), jnp.float32)

    npad = EPAD - E
    pad_ids = jax.lax.iota(jnp.int32, npad)
    src_p = jnp.concatenate([src, pad_ids % N])
    dst_p = jnp.concatenate([dst, N + pad_ids % (NP - N)])

    deg2 = _deg_call(dst, ones_ch, zrow)
    dega = deg2[0].reshape(GRID, 1, RB)
    degb = deg2[1].reshape(GRID, 1, RB)
    y = _tc1_call(x, W1, dega, degb)
    acc2 = _msg_call(y, src_p, dst_p, zblk)
    out = _tc2_call(acc2, y, x, dega, degb,
                    b1.reshape(1, D), ln_gamma.reshape(1, D),
                    ln_beta.reshape(1, D), Wfc, bfc.reshape(1, C))
    return out
